# baseline (speedup 1.0000x reference)
import functools

import jax
import jax.numpy as jnp
from jax import lax
from jax.experimental import pallas as pl
from jax.experimental.pallas import tpu as pltpu
from jax.experimental.pallas import tpu_sc as plsc

NC = 2
NS = 16
L = 16
CH = 128
HALF = 128


def _emb_body(x_ref, w_ref, o_ref):
    h = jnp.dot(x_ref[...], w_ref[...], preferred_element_type=jnp.float32)
    o_ref[0] = h[:, :HALF]
    o_ref[1] = h[:, HALF:]


def _gconv_body(h_ref, a_ref, w_ref, b_ref, o_ref):
    h0 = h_ref[0]
    h1 = h_ref[1]
    z = (jnp.dot(h0, w_ref[0:HALF], preferred_element_type=jnp.float32)
         + jnp.dot(h1, w_ref[HALF:2 * HALF], preferred_element_type=jnp.float32)
         + jnp.dot(a_ref[0], w_ref[2 * HALF:3 * HALF], preferred_element_type=jnp.float32)
         + jnp.dot(a_ref[1], w_ref[3 * HALF:4 * HALF], preferred_element_type=jnp.float32)
         + b_ref[...])
    hn = jnp.maximum(z, 0.0)
    o_ref[0] = h0 + hn[:, :HALF]
    o_ref[1] = h1 + hn[:, HALF:]


def _final_body(h_ref, w_ref, o_ref):
    z = (jnp.dot(h_ref[0], w_ref[0:HALF], preferred_element_type=jnp.float32)
         + jnp.dot(h_ref[1], w_ref[HALF:2 * HALF], preferred_element_type=jnp.float32))
    s = jnp.sum(z * z, axis=1, keepdims=True)
    o_ref[...] = z / jnp.sqrt(s)


def _emb_call(x, w_emb, block_rows):
    n, d_in = x.shape
    grid = n // block_rows
    return pl.pallas_call(
        _emb_body,
        grid=(grid,),
        in_specs=[
            pl.BlockSpec((block_rows, d_in), lambda i: (i, 0)),
            pl.BlockSpec((d_in, 2 * HALF), lambda i: (0, 0)),
        ],
        out_specs=pl.BlockSpec((2, block_rows, HALF), lambda i: (0, i, 0)),
        out_shape=jax.ShapeDtypeStruct((2, n, HALF), jnp.float32),
    )(x, w_emb)


def _gconv_call(h, agg, w, b, block_rows):
    n = h.shape[1]
    grid = n // block_rows
    return pl.pallas_call(
        _gconv_body,
        grid=(grid,),
        in_specs=[
            pl.BlockSpec((2, block_rows, HALF), lambda i: (0, i, 0)),
            pl.BlockSpec((2, block_rows, HALF), lambda i: (0, i, 0)),
            pl.BlockSpec((4 * HALF, 2 * HALF), lambda i: (0, 0)),
            pl.BlockSpec((1, 2 * HALF), lambda i: (0, 0)),
        ],
        out_specs=pl.BlockSpec((2, block_rows, HALF), lambda i: (0, i, 0)),
        out_shape=jax.ShapeDtypeStruct((2, n, HALF), jnp.float32),
    )(h, agg, w, b)


def _final_call(h, w_last, block_rows):
    n = h.shape[1]
    grid = n // block_rows
    return pl.pallas_call(
        _final_body,
        grid=(grid,),
        in_specs=[
            pl.BlockSpec((2, block_rows, HALF), lambda i: (0, i, 0)),
            pl.BlockSpec((2 * HALF, HALF), lambda i: (0, 0)),
        ],
        out_specs=pl.BlockSpec((block_rows, HALF), lambda i: (i, 0)),
        out_shape=jax.ShapeDtypeStruct((n, HALF), jnp.float32),
    )(h, w_last)


@functools.lru_cache(maxsize=None)
def _make_spmm(n, n_pad, nchunks):
    rows_per_tile = n_pad // NS
    nfull = rows_per_tile // CH
    assert nfull * CH == rows_per_tile
    mesh = plsc.VectorSubcoreMesh(core_axis_name="c", subcore_axis_name="s")

    @functools.partial(
        pl.kernel,
        out_type=jax.ShapeDtypeStruct((NC * n_pad, HALF), jnp.float32),
        mesh=mesh,
        scratch_types=[
            pltpu.VMEM((2, 2, CH), jnp.int32),
            pltpu.VMEM((2, CH), jnp.float32),
            pltpu.VMEM((2, CH, HALF), jnp.float32),
            pltpu.VMEM_SHARED((n_pad, HALF), jnp.float32),
            pltpu.SemaphoreType.DMA,
            pltpu.SemaphoreType.DMA,
            pltpu.SemaphoreType.DMA,
            pltpu.SemaphoreType.DMA,
        ],
    )
    def spmm(h_hbm, rec_hbm, w_hbm, out_hbm,
             rec_v, w_v, rows_v, acc,
             si0, si1, sg0, sg1):
        c = lax.axis_index("c")
        s = lax.axis_index("s")
        wid = c * NS + s
        sem_i = (si0, si1)
        sem_g = (sg0, sg1)

        def issue_idx(g, k):
            pltpu.async_copy(rec_hbm.at[wid, g], rec_v.at[k], sem_i[k])
            woff = (s * nchunks + g) * CH
            pltpu.async_copy(
                w_hbm.at[pl.ds(pl.multiple_of(woff, 8), CH)],
                w_v.at[k], sem_i[k])

        def wait_idx(k):
            pltpu.make_async_copy(rec_hbm.at[0, 0], rec_v.at[k], sem_i[k]).wait()
            pltpu.make_async_copy(w_hbm.at[pl.ds(0, CH)], w_v.at[k],
                                  sem_i[k]).wait()

        def issue_gather_w(g, k):
            pltpu.async_copy(h_hbm.at[rec_v.at[k, 0]], rows_v.at[k], sem_g[k])

        def wait_gather_w(k):
            pltpu.make_async_copy(h_hbm.at[rec_v.at[0, 0]], rows_v.at[k],
                                  sem_g[k]).wait()

        def scale_scatter(k):
            def group(gr, _):
                base = pl.multiple_of(gr * L, L)
                wgrp = w_v[k, pl.ds(base, L)]
                for ii in range(L):
                    wsplat = lax.gather(
                        wgrp,
                        jnp.full((L, 1), ii, jnp.int32),
                        lax.GatherDimensionNumbers(
                            offset_dims=(), collapsed_slice_dims=(0,),
                            start_index_map=(0,)),
                        slice_sizes=(1,),
                        mode=lax.GatherScatterMode.PROMISE_IN_BOUNDS)
                    for q in range(HALF // L):
                        sl = pl.ds(q * L, L)
                        rows_v[k, base + ii, sl] = rows_v[k, base + ii, sl] * wsplat
                return 0
            lax.fori_loop(0, CH // L, group, 0)
            pltpu.sync_copy(rows_v.at[k], acc.at[rec_v.at[k, 1]], add=True)

        def zrow(i, _):
            for q in range(HALF // L):
                rows_v[0, i, pl.ds(q * L, L)] = jnp.zeros((L,), jnp.float32)
            return 0
        lax.fori_loop(0, CH, zrow, 0)

        base = pl.multiple_of(s * rows_per_tile, CH)

        def zacc(t, _):
            pltpu.sync_copy(rows_v.at[0],
                            acc.at[pl.ds(pl.multiple_of(base + t * CH, CH), CH)])
            return 0
        lax.fori_loop(0, nfull, zacc, 0)
        plsc.subcore_barrier()

        npairs = nchunks // 2
        pltpu.sync_copy(rec_hbm.at[wid, 0], rec_v.at[0])
        woff0 = pl.multiple_of(s * nchunks * CH, 8)
        pltpu.sync_copy(w_hbm.at[pl.ds(woff0, CH)], w_v.at[0])
        issue_gather_w(0, 0)
        issue_idx(1, 1)

        def pair(t, _):
            g = pl.multiple_of(t * 2, 2)
            wait_gather_w(0)
            wait_idx(1)
            issue_gather_w(g + 1, 1)
            scale_scatter(0)

            @pl.when(g + 2 < nchunks)
            def _():
                issue_idx(g + 2, 0)

            wait_gather_w(1)

            @pl.when(g + 2 < nchunks)
            def _():
                wait_idx(0)
                issue_gather_w(g + 2, 0)
            scale_scatter(1)

            @pl.when(g + 3 < nchunks)
            def _():
                issue_idx(g + 3, 1)
            return 0
        lax.fori_loop(0, npairs, pair, 0)
        plsc.subcore_barrier()

        obase = pl.multiple_of(c * n_pad + base, CH)

        def wout(t, _):
            pltpu.sync_copy(acc.at[pl.ds(pl.multiple_of(base + t * CH, CH), CH)], rows_v.at[0])
            pltpu.sync_copy(rows_v.at[0], out_hbm.at[pl.ds(pl.multiple_of(obase + t * CH, CH), CH)])
            return 0
        lax.fori_loop(0, nfull, wout, 0)

    return spmm


def kernel(x, edge_index, edge_weight, g_size, W_emb, W_gc1, b_gc1,
           W_gc2, b_gc2, W_last):
    n = x.shape[0]
    e = edge_weight.shape[0]

    ew = -(-e // NS)
    nchunks = -(-ew // CH)
    nchunks += nchunks % 2
    ew = nchunks * CH
    pad = NS * ew - e
    src = jnp.concatenate([edge_index[1], jnp.zeros((pad,), jnp.int32)])
    dst = jnp.concatenate([edge_index[0], jnp.zeros((pad,), jnp.int32)])
    w = jnp.concatenate([edge_weight, jnp.zeros((pad,), jnp.float32)])
    src_r = src.reshape(NS, nchunks, CH)
    dst_r = dst.reshape(NS, nchunks, CH)
    rec0 = jnp.stack([src_r, dst_r], axis=2)
    rec1 = jnp.stack([src_r + n, dst_r], axis=2)
    rec_hbm = jnp.concatenate([rec0, rec1], axis=0)

    n_pad = -(-n // (NS * CH)) * (NS * CH)
    block_rows = 2000
    spmm = _make_spmm(n, n_pad, nchunks)

    h = _emb_call(x, W_emb, block_rows)
    for W, b in ((W_gc1, b_gc1), (W_gc2, b_gc2)):
        agg = spmm(h.reshape(2 * n, HALF), rec_hbm, w)
        h = _gconv_call(h, agg.reshape(2, n_pad, HALF), W,
                        b.reshape(1, 2 * HALF), block_rows)
    return _final_call(h, W_last, block_rows)


# speedup vs baseline: 1.0337x; 1.0337x over previous
import functools

import jax
import jax.numpy as jnp
from jax import lax
from jax.experimental import pallas as pl
from jax.experimental.pallas import tpu as pltpu
from jax.experimental.pallas import tpu_sc as plsc

NC = 2
NS = 16
L = 16
CH = 128
HALF = 128


def _emb_body(x_ref, w_ref, o_ref):
    h = jnp.dot(x_ref[...], w_ref[...], preferred_element_type=jnp.float32)
    o_ref[0] = h[:, :HALF]
    o_ref[1] = h[:, HALF:]


def _gconv_body(h_ref, a_ref, w_ref, b_ref, o_ref):
    h0 = h_ref[0]
    h1 = h_ref[1]
    z = (jnp.dot(h0, w_ref[0:HALF], preferred_element_type=jnp.float32)
         + jnp.dot(h1, w_ref[HALF:2 * HALF], preferred_element_type=jnp.float32)
         + jnp.dot(a_ref[0], w_ref[2 * HALF:3 * HALF], preferred_element_type=jnp.float32)
         + jnp.dot(a_ref[1], w_ref[3 * HALF:4 * HALF], preferred_element_type=jnp.float32)
         + b_ref[...])
    hn = jnp.maximum(z, 0.0)
    o_ref[0] = h0 + hn[:, :HALF]
    o_ref[1] = h1 + hn[:, HALF:]


def _final_body(h_ref, w_ref, o_ref):
    z = (jnp.dot(h_ref[0], w_ref[0:HALF], preferred_element_type=jnp.float32)
         + jnp.dot(h_ref[1], w_ref[HALF:2 * HALF], preferred_element_type=jnp.float32))
    s = jnp.sum(z * z, axis=1, keepdims=True)
    o_ref[...] = z / jnp.sqrt(s)


def _emb_call(x, w_emb, block_rows):
    n, d_in = x.shape
    grid = n // block_rows
    return pl.pallas_call(
        _emb_body,
        grid=(grid,),
        in_specs=[
            pl.BlockSpec((block_rows, d_in), lambda i: (i, 0)),
            pl.BlockSpec((d_in, 2 * HALF), lambda i: (0, 0)),
        ],
        out_specs=pl.BlockSpec((2, block_rows, HALF), lambda i: (0, i, 0)),
        out_shape=jax.ShapeDtypeStruct((2, n, HALF), jnp.float32),
    )(x, w_emb)


def _gconv_call(h, agg, w, b, block_rows):
    n = h.shape[1]
    grid = n // block_rows
    return pl.pallas_call(
        _gconv_body,
        grid=(grid,),
        in_specs=[
            pl.BlockSpec((2, block_rows, HALF), lambda i: (0, i, 0)),
            pl.BlockSpec((2, block_rows, HALF), lambda i: (0, i, 0)),
            pl.BlockSpec((4 * HALF, 2 * HALF), lambda i: (0, 0)),
            pl.BlockSpec((1, 2 * HALF), lambda i: (0, 0)),
        ],
        out_specs=pl.BlockSpec((2, block_rows, HALF), lambda i: (0, i, 0)),
        out_shape=jax.ShapeDtypeStruct((2, n, HALF), jnp.float32),
    )(h, agg, w, b)


def _final_call(h, w_last, block_rows):
    n = h.shape[1]
    grid = n // block_rows
    return pl.pallas_call(
        _final_body,
        grid=(grid,),
        in_specs=[
            pl.BlockSpec((2, block_rows, HALF), lambda i: (0, i, 0)),
            pl.BlockSpec((2 * HALF, HALF), lambda i: (0, 0)),
        ],
        out_specs=pl.BlockSpec((block_rows, HALF), lambda i: (i, 0)),
        out_shape=jax.ShapeDtypeStruct((n, HALF), jnp.float32),
    )(h, w_last)


@functools.lru_cache(maxsize=None)
def _make_spmm(n, n_pad, nchunks):
    rows_per_tile = n_pad // NS
    nfull = rows_per_tile // CH
    assert nfull * CH == rows_per_tile
    mesh = plsc.VectorSubcoreMesh(core_axis_name="c", subcore_axis_name="s")

    @functools.partial(
        pl.kernel,
        out_type=jax.ShapeDtypeStruct((NC * n_pad, HALF), jnp.float32),
        mesh=mesh,
        scratch_types=[
            pltpu.VMEM((2, CH), jnp.int32),
            pltpu.VMEM((2, CH), jnp.int32),
            pltpu.VMEM((2, CH), jnp.float32),
            pltpu.VMEM((2, CH, HALF), jnp.float32),
            pltpu.VMEM_SHARED((n_pad, HALF), jnp.float32),
            pltpu.SemaphoreType.DMA,
            pltpu.SemaphoreType.DMA,
            pltpu.SemaphoreType.DMA,
            pltpu.SemaphoreType.DMA,
            pltpu.SemaphoreType.DMA,
            pltpu.SemaphoreType.DMA,
        ],
    )
    def spmm(h_hbm, src_hbm, dst_hbm, w_hbm, out_hbm,
             src_v, dst_v, w_v, rows_v, acc,
             si0, si1, sg0, sg1, sw0, sw1):
        c = lax.axis_index("c")
        s = lax.axis_index("s")
        wid = c * NS + s
        sem_i = (si0, si1)
        sem_g = (sg0, sg1)
        sem_w = (sw0, sw1)

        def issue_idx(g, k):
            pltpu.async_copy(src_hbm.at[wid, g], src_v.at[k], sem_i[k])
            pltpu.async_copy(dst_hbm.at[s, g], dst_v.at[k], sem_i[k])

        def wait_idx(k):
            pltpu.make_async_copy(src_hbm.at[0, 0], src_v.at[k], sem_i[k]).wait()
            pltpu.make_async_copy(dst_hbm.at[0, 0], dst_v.at[k], sem_i[k]).wait()

        def issue_gather_w(g, k):
            pltpu.async_copy(h_hbm.at[src_v.at[k]], rows_v.at[k], sem_g[k])
            woff = (s * nchunks + g) * CH
            pltpu.async_copy(
                w_hbm.at[pl.ds(pl.multiple_of(woff, 8), CH)],
                w_v.at[k], sem_w[k])

        def wait_gather_w(k):
            pltpu.make_async_copy(h_hbm.at[src_v.at[0]], rows_v.at[k],
                                  sem_g[k]).wait()
            pltpu.make_async_copy(w_hbm.at[pl.ds(0, CH)], w_v.at[k],
                                  sem_w[k]).wait()

        def scale_scatter(k):
            def group(gr, _):
                base = pl.multiple_of(gr * L, L)
                wgrp = w_v[k, pl.ds(base, L)]
                for ii in range(L):
                    wsplat = lax.gather(
                        wgrp,
                        jnp.full((L, 1), ii, jnp.int32),
                        lax.GatherDimensionNumbers(
                            offset_dims=(), collapsed_slice_dims=(0,),
                            start_index_map=(0,)),
                        slice_sizes=(1,),
                        mode=lax.GatherScatterMode.PROMISE_IN_BOUNDS)
                    for q in range(HALF // L):
                        sl = pl.ds(q * L, L)
                        rows_v[k, base + ii, sl] = rows_v[k, base + ii, sl] * wsplat
                return 0
            lax.fori_loop(0, CH // L, group, 0)
            pltpu.sync_copy(rows_v.at[k], acc.at[dst_v.at[k]], add=True)

        def zrow(i, _):
            for q in range(HALF // L):
                rows_v[0, i, pl.ds(q * L, L)] = jnp.zeros((L,), jnp.float32)
            return 0
        lax.fori_loop(0, CH, zrow, 0)

        base = pl.multiple_of(s * rows_per_tile, CH)

        def zacc(t, _):
            pltpu.sync_copy(rows_v.at[0],
                            acc.at[pl.ds(pl.multiple_of(base + t * CH, CH), CH)])
            return 0
        lax.fori_loop(0, nfull, zacc, 0)
        plsc.subcore_barrier()

        npairs = nchunks // 2
        pltpu.sync_copy(src_hbm.at[wid, 0], src_v.at[0])
        pltpu.sync_copy(dst_hbm.at[s, 0], dst_v.at[0])
        issue_gather_w(0, 0)
        issue_idx(1, 1)

        def pair(t, _):
            g = pl.multiple_of(t * 2, 2)
            wait_gather_w(0)
            wait_idx(1)
            issue_gather_w(g + 1, 1)
            scale_scatter(0)

            @pl.when(g + 2 < nchunks)
            def _():
                issue_idx(g + 2, 0)

            wait_gather_w(1)

            @pl.when(g + 2 < nchunks)
            def _():
                wait_idx(0)
                issue_gather_w(g + 2, 0)
            scale_scatter(1)

            @pl.when(g + 3 < nchunks)
            def _():
                issue_idx(g + 3, 1)
            return 0
        lax.fori_loop(0, npairs, pair, 0)
        plsc.subcore_barrier()

        obase = pl.multiple_of(c * n_pad + base, CH)

        def wout(t, _):
            pltpu.sync_copy(acc.at[pl.ds(pl.multiple_of(base + t * CH, CH), CH)], rows_v.at[0])
            pltpu.sync_copy(rows_v.at[0], out_hbm.at[pl.ds(pl.multiple_of(obase + t * CH, CH), CH)])
            return 0
        lax.fori_loop(0, nfull, wout, 0)

    return spmm


def kernel(x, edge_index, edge_weight, g_size, W_emb, W_gc1, b_gc1,
           W_gc2, b_gc2, W_last):
    n = x.shape[0]
    e = edge_weight.shape[0]

    ew = -(-e // NS)
    nchunks = -(-ew // CH)
    nchunks += nchunks % 2
    ew = nchunks * CH
    pad = NS * ew - e
    src = jnp.concatenate([edge_index[1], jnp.zeros((pad,), jnp.int32)])
    dst = jnp.concatenate([edge_index[0], jnp.zeros((pad,), jnp.int32)])
    w = jnp.concatenate([edge_weight, jnp.zeros((pad,), jnp.float32)])
    src_r = src.reshape(NS, nchunks, CH)
    src_hbm = jnp.concatenate([src_r, src_r + n], axis=0)
    dst_hbm = dst.reshape(NS, nchunks, CH)

    n_pad = -(-n // (NS * CH)) * (NS * CH)
    block_rows = 2000
    spmm = _make_spmm(n, n_pad, nchunks)

    h = _emb_call(x, W_emb, block_rows)
    for W, b in ((W_gc1, b_gc1), (W_gc2, b_gc2)):
        agg = spmm(h.reshape(2 * n, HALF), src_hbm, dst_hbm, w)
        h = _gconv_call(h, agg.reshape(2, n_pad, HALF), W,
                        b.reshape(1, 2 * HALF), block_rows)
    return _final_call(h, W_last, block_rows)
